# Initial kernel scaffold; baseline (speedup 1.0000x reference)
import jax
import jax.numpy as jnp
from jax.experimental import pallas as pl


def kernel(x, edge_index, batch, W1, b1, W2, b2, W3, b3, W4, b4, W5, b5, W6, b6, W7, b7, W8, b8, W9, b9, W10, b10, W11, b11, W12, b12):
    raise NotImplementedError("write your pallas kernel here")


# speedup vs baseline: 5.7249x; 5.7249x over previous
import functools

import jax
import jax.numpy as jnp
from jax import lax
from jax.experimental import pallas as pl
from jax.experimental.pallas import tpu as pltpu
from jax.experimental.pallas import tpu_sc as plsc

_N = 10000
_NP = 10240
_E = 160000
_EP = 163840
_NC = 2
_NS = 16
_EPW = _EP // (_NC * _NS)
_C = 128
_RPT = _NP // _NS
_BR = 512


def _sc_aggregate(table, src, dst, w):
    mesh = plsc.VectorSubcoreMesh(core_axis_name="c", subcore_axis_name="s")

    @functools.partial(
        pl.kernel,
        out_type=jax.ShapeDtypeStruct((_NC, _NP, w), jnp.float32),
        mesh=mesh,
        scratch_types=[
            pltpu.VMEM((_C,), jnp.int32),
            pltpu.VMEM((_C,), jnp.int32),
            pltpu.VMEM((_C, w), jnp.float32),
            pltpu.VMEM((_C, w), jnp.float32),
            pltpu.VMEM_SHARED((_NP, w), jnp.float32),
            pltpu.SemaphoreType.DMA,
        ],
        compiler_params=pltpu.CompilerParams(use_tc_tiling_on_sc=False),
    )
    def agg(table_h, src_h, dst_h, zero_h, out_h, sidx, didx, rows, zbuf, acc, sem):
        cid = lax.axis_index("c")
        sid = lax.axis_index("s")
        wid = cid * _NS + sid

        pltpu.sync_copy(zero_h, zbuf)
        for k in range(_RPT // _C):
            off = pl.multiple_of(sid * _RPT + k * _C, _C)
            pltpu.sync_copy(zbuf, acc.at[pl.ds(off, _C)])
        plsc.subcore_barrier()

        ebase = pl.multiple_of(wid * _EPW, _C)

        def chunk(k, carry):
            base = pl.multiple_of(ebase + k * _C, _C)
            pltpu.sync_copy(src_h.at[pl.ds(base, _C)], sidx)
            pltpu.sync_copy(dst_h.at[pl.ds(base, _C)], didx)
            pltpu.async_copy(table_h.at[sidx], rows, sem).wait()
            pltpu.sync_copy(rows, acc.at[didx], add=True)
            return carry

        lax.fori_loop(0, _EPW // _C, chunk, 0)
        plsc.subcore_barrier()

        for k in range(_RPT // _C):
            off = pl.multiple_of(sid * _RPT + k * _C, _C)
            pltpu.sync_copy(acc.at[pl.ds(off, _C)], zbuf)
            pltpu.sync_copy(zbuf, out_h.at[cid, pl.ds(off, _C)])

    return agg(table, src, dst, jnp.zeros((_C, w), jnp.float32))


def _rows_call(fn, row_ins, whole_ins, out_widths):
    nb = _NP // _BR
    nri, nwi = len(row_ins), len(whole_ins)
    in_specs = (
        [pl.BlockSpec((_BR, a.shape[1]), lambda i: (i, 0)) for a in row_ins]
        + [pl.BlockSpec(a.shape, lambda i, _nd=a.ndim: (0,) * _nd) for a in whole_ins]
    )
    out_specs = [pl.BlockSpec((_BR, w), lambda i: (i, 0)) for w in out_widths]
    out_shape = [jax.ShapeDtypeStruct((_NP, w), jnp.float32) for w in out_widths]

    def body(*refs):
        ins = [r[...] for r in refs[: nri + nwi]]
        outs = fn(*ins)
        if not isinstance(outs, (tuple, list)):
            outs = (outs,)
        for r, o in zip(refs[nri + nwi:], outs):
            r[...] = o

    return pl.pallas_call(
        body,
        grid=(nb,),
        in_specs=in_specs,
        out_specs=out_specs,
        out_shape=out_shape,
    )(*row_ins, *whole_ins)


def _matmul(a, w):
    return lax.dot_general(a, w, (((1,), (0,)), ((), ())),
                           preferred_element_type=jnp.float32)


def kernel(x, edge_index, batch,
           W1, b1, W2, b2, W3, b3, W4, b4, W5, b5, W6, b6,
           W7, b7, W8, b8, W9, b9, W10, b10, W11, b11, W12, b12):
    Ws = [W1, W2, W3, W4, W5, W6, W7, W8, W9, W10, W11, W12]
    bs = [b.reshape(1, -1) for b in
          [b1, b2, b3, b4, b5, b6, b7, b8, b9, b10, b11, b12]]

    sentinel = jnp.full((_EP - _E,), _N, jnp.int32)
    srcp = jnp.concatenate([edge_index[0], sentinel])
    dstp = jnp.concatenate([edge_index[1], sentinel])
    xp = jnp.pad(x, ((0, _NP - _N), (0, 0)))

    P = _sc_aggregate(jnp.ones((_NP, 8), jnp.float32), srcp, dstp, 8)

    def s0(p0, p1, xb):
        dinv = lax.rsqrt(p0[:, :1] + p1[:, :1] + 1.0)
        return jnp.broadcast_to(dinv, (dinv.shape[0], 8)), dinv * xb

    dinv8, u = _rows_call(s0, [P[0], P[1], xp], [], [8, 8])

    for i in range(5):
        P = _sc_aggregate(u, srcp, dstp, u.shape[1])

        def sexp(p0, p1, ub, dv, W, b):
            dvc = dv[:, :1]
            h = jnp.tanh(_matmul(dvc * (p0 + p1 + ub), W) + b)
            return dvc * h

        u = _rows_call(sexp, [P[0], P[1], u, dinv8], [Ws[i], bs[i]],
                       [Ws[i].shape[1]])[0]

    u6L, u6R = u[:, :128], u[:, 128:]
    PL = _sc_aggregate(u6L, srcp, dstp, 128)
    PR = _sc_aggregate(u6R, srcp, dstp, 128)
    W7a, W7b = W7[:512], W7[512:]

    def s6(p0l, p1l, p0r, p1r, ul, ur, dv, xb, W6_, b6_, W7a_, W7b_):
        dvc = dv[:, :1]
        z = jnp.concatenate([dvc * (p0l + p1l + ul), dvc * (p0r + p1r + ur)],
                            axis=1)
        h6 = jnp.tanh(_matmul(z, W6_) + b6_)
        g7 = dvc * (_matmul(h6, W7a_) + _matmul(xb, W7b_))
        return g7[:, :128], g7[:, 128:]

    g7L, g7R = _rows_call(s6, [PL[0], PL[1], PR[0], PR[1], u6L, u6R, dinv8, xp],
                          [Ws[5], bs[5], W7a, W7b], [128, 128])

    PL = _sc_aggregate(g7L, srcp, dstp, 128)
    PR = _sc_aggregate(g7R, srcp, dstp, 128)

    def s7(p0l, p1l, p0r, p1r, gl, gr, dv, W8_, b7_):
        dvc = dv[:, :1]
        conv = jnp.concatenate([dvc * (p0l + p1l + gl),
                                dvc * (p0r + p1r + gr)], axis=1) + b7_
        return dvc * _matmul(jnp.tanh(conv), W8_)

    g = _rows_call(s7, [PL[0], PL[1], PR[0], PR[1], g7L, g7R, dinv8],
                   [Ws[7], bs[6]], [128])[0]

    for L in range(8, 12):
        P = _sc_aggregate(g, srcp, dstp, g.shape[1])

        def scon(p0, p1, gb, dv, Wn, bl):
            dvc = dv[:, :1]
            h = jnp.tanh(dvc * (p0 + p1 + gb) + bl)
            return dvc * _matmul(h, Wn)

        g = _rows_call(scon, [P[0], P[1], g, dinv8], [Ws[L], bs[L - 1]],
                       [Ws[L].shape[1]])[0]

    P = _sc_aggregate(g, srcp, dstp, 8)

    def sfin(p0, p1, gb, dv, bl):
        return dv[:, :1] * (p0 + p1 + gb) + bl

    out = _rows_call(sfin, [P[0], P[1], g, dinv8], [bs[11]], [8])[0]
    return out[:_N]


# speedup vs baseline: 7.3789x; 1.2889x over previous
import functools

import jax
import jax.numpy as jnp
from jax import lax
from jax.experimental import pallas as pl
from jax.experimental.pallas import tpu as pltpu
from jax.experimental.pallas import tpu_sc as plsc

_N = 10000
_NP = 10240
_E = 160000
_EP = 163840
_NC = 2
_NS = 16
_EPW = _EP // (_NC * _NS)
_C = 128
_RPT = _NP // _NS
_BR = 512


_NB = 2
_CPW = _EPW // _C
_NR = _CPW // _NB


def _sc_aggregate(table, src2d, dst2d, w):
    mesh = plsc.VectorSubcoreMesh(core_axis_name="c", subcore_axis_name="s")

    @functools.partial(
        pl.kernel,
        out_type=jax.ShapeDtypeStruct((_NC, _NP, w), jnp.float32),
        mesh=mesh,
        scratch_types=[
            pltpu.VMEM((_CPW, _C), jnp.int32),
            pltpu.VMEM((_CPW, _C), jnp.int32),
            pltpu.VMEM((_C, w), jnp.float32),
            pltpu.VMEM((_C, w), jnp.float32),
            pltpu.VMEM_SHARED((_NP, w), jnp.float32),
            *([pltpu.SemaphoreType.DMA] * (2 * _NB)),
        ],
        compiler_params=pltpu.CompilerParams(use_tc_tiling_on_sc=False),
    )
    def agg(table_h, src_h, dst_h, zero_h, out_h,
            sidx, didx, r0, r1, acc, g0, g1, s0, s1):
        rows = [r0, r1]
        gsem = [g0, g1]
        ssem = [s0, s1]
        cid = lax.axis_index("c")
        sid = lax.axis_index("s")
        wid = cid * _NS + sid

        ibase = pl.multiple_of(wid * _CPW, 8)
        pltpu.sync_copy(src_h.at[pl.ds(ibase, _CPW)], sidx)
        pltpu.sync_copy(dst_h.at[pl.ds(ibase, _CPW)], didx)

        pltpu.sync_copy(zero_h, rows[0])
        for k in range(_RPT // _C):
            off = pl.multiple_of(sid * _RPT + k * _C, _C)
            pltpu.sync_copy(rows[0], acc.at[pl.ds(off, _C)])
        for b in range(_NB):
            pltpu.async_copy(table_h.at[sidx.at[b]], rows[b], gsem[b])
        plsc.subcore_barrier()

        def rnd(r, carry):
            for b in range(_NB):
                k = r * _NB + b
                pltpu.make_async_copy(table_h.at[sidx.at[k]], rows[b],
                                      gsem[b]).wait()
                pltpu.async_copy(rows[b], acc.at[didx.at[k]], ssem[b],
                                 add=True)
            for b in range(_NB):
                k = (r + 1) * _NB + b
                pltpu.make_async_copy(rows[b], acc.at[didx.at[k - _NB]],
                                      ssem[b]).wait()
                pltpu.async_copy(table_h.at[sidx.at[k]], rows[b], gsem[b])
            return carry

        lax.fori_loop(0, _NR - 1, rnd, 0)
        for b in range(_NB):
            k = (_NR - 1) * _NB + b
            pltpu.make_async_copy(table_h.at[sidx.at[k]], rows[b],
                                  gsem[b]).wait()
            pltpu.async_copy(rows[b], acc.at[didx.at[k]], ssem[b], add=True)
        for b in range(_NB):
            k = (_NR - 1) * _NB + b
            pltpu.make_async_copy(rows[b], acc.at[didx.at[k]],
                                  ssem[b]).wait()
        plsc.subcore_barrier()

        nwb = _RPT // _C
        for k in range(nwb):
            off = pl.multiple_of(sid * _RPT + k * _C, _C)
            b = k % _NB
            if k >= _NB:
                poff = pl.multiple_of(sid * _RPT + (k - _NB) * _C, _C)
                pltpu.make_async_copy(rows[b], out_h.at[cid, pl.ds(poff, _C)],
                                      ssem[b]).wait()
            pltpu.async_copy(acc.at[pl.ds(off, _C)], rows[b], gsem[b]).wait()
            pltpu.async_copy(rows[b], out_h.at[cid, pl.ds(off, _C)], ssem[b])
        for k in range(nwb - _NB, nwb):
            off = pl.multiple_of(sid * _RPT + k * _C, _C)
            pltpu.make_async_copy(rows[k % _NB], out_h.at[cid, pl.ds(off, _C)],
                                  ssem[k % _NB]).wait()

    return agg(table, src2d, dst2d, jnp.zeros((_C, w), jnp.float32))


def _rows_call(fn, row_ins, whole_ins, out_widths):
    nb = _NP // _BR
    nri, nwi = len(row_ins), len(whole_ins)
    in_specs = (
        [pl.BlockSpec((_BR, a.shape[1]), lambda i: (i, 0)) for a in row_ins]
        + [pl.BlockSpec(a.shape, lambda i, _nd=a.ndim: (0,) * _nd) for a in whole_ins]
    )
    out_specs = [pl.BlockSpec((_BR, w), lambda i: (i, 0)) for w in out_widths]
    out_shape = [jax.ShapeDtypeStruct((_NP, w), jnp.float32) for w in out_widths]

    def body(*refs):
        ins = [r[...] for r in refs[: nri + nwi]]
        outs = fn(*ins)
        if not isinstance(outs, (tuple, list)):
            outs = (outs,)
        for r, o in zip(refs[nri + nwi:], outs):
            r[...] = o

    return pl.pallas_call(
        body,
        grid=(nb,),
        in_specs=in_specs,
        out_specs=out_specs,
        out_shape=out_shape,
    )(*row_ins, *whole_ins)


def _matmul(a, w):
    return lax.dot_general(a, w, (((1,), (0,)), ((), ())),
                           preferred_element_type=jnp.float32)


def kernel(x, edge_index, batch,
           W1, b1, W2, b2, W3, b3, W4, b4, W5, b5, W6, b6,
           W7, b7, W8, b8, W9, b9, W10, b10, W11, b11, W12, b12):
    Ws = [W1, W2, W3, W4, W5, W6, W7, W8, W9, W10, W11, W12]
    bs = [b.reshape(1, -1) for b in
          [b1, b2, b3, b4, b5, b6, b7, b8, b9, b10, b11, b12]]

    sentinel = jnp.full((_EP - _E,), _N, jnp.int32)
    srcp = jnp.concatenate([edge_index[0], sentinel]).reshape(_EP // _C, _C)
    dstp = jnp.concatenate([edge_index[1], sentinel]).reshape(_EP // _C, _C)
    xp = jnp.pad(x, ((0, _NP - _N), (0, 0)))

    P = _sc_aggregate(jnp.ones((_NP, 8), jnp.float32), srcp, dstp, 8)

    def s0(p0, p1, xb):
        dinv = lax.rsqrt(p0[:, :1] + p1[:, :1] + 1.0)
        return jnp.broadcast_to(dinv, (dinv.shape[0], 8)), dinv * xb

    dinv8, u = _rows_call(s0, [P[0], P[1], xp], [], [8, 8])

    for i in range(5):
        P = _sc_aggregate(u, srcp, dstp, u.shape[1])

        def sexp(p0, p1, ub, dv, W, b):
            dvc = dv[:, :1]
            h = jnp.tanh(_matmul(dvc * (p0 + p1 + ub), W) + b)
            return dvc * h

        u = _rows_call(sexp, [P[0], P[1], u, dinv8], [Ws[i], bs[i]],
                       [Ws[i].shape[1]])[0]

    u6L, u6R = u[:, :128], u[:, 128:]
    PL = _sc_aggregate(u6L, srcp, dstp, 128)
    PR = _sc_aggregate(u6R, srcp, dstp, 128)
    W7a, W7b = W7[:512], W7[512:]

    def s6(p0l, p1l, p0r, p1r, ul, ur, dv, xb, W6_, b6_, W7a_, W7b_):
        dvc = dv[:, :1]
        z = jnp.concatenate([dvc * (p0l + p1l + ul), dvc * (p0r + p1r + ur)],
                            axis=1)
        h6 = jnp.tanh(_matmul(z, W6_) + b6_)
        g7 = dvc * (_matmul(h6, W7a_) + _matmul(xb, W7b_))
        return g7[:, :128], g7[:, 128:]

    g7L, g7R = _rows_call(s6, [PL[0], PL[1], PR[0], PR[1], u6L, u6R, dinv8, xp],
                          [Ws[5], bs[5], W7a, W7b], [128, 128])

    PL = _sc_aggregate(g7L, srcp, dstp, 128)
    PR = _sc_aggregate(g7R, srcp, dstp, 128)

    def s7(p0l, p1l, p0r, p1r, gl, gr, dv, W8_, b7_):
        dvc = dv[:, :1]
        conv = jnp.concatenate([dvc * (p0l + p1l + gl),
                                dvc * (p0r + p1r + gr)], axis=1) + b7_
        return dvc * _matmul(jnp.tanh(conv), W8_)

    g = _rows_call(s7, [PL[0], PL[1], PR[0], PR[1], g7L, g7R, dinv8],
                   [Ws[7], bs[6]], [128])[0]

    for L in range(8, 12):
        P = _sc_aggregate(g, srcp, dstp, g.shape[1])

        def scon(p0, p1, gb, dv, Wn, bl):
            dvc = dv[:, :1]
            h = jnp.tanh(dvc * (p0 + p1 + gb) + bl)
            return dvc * _matmul(h, Wn)

        g = _rows_call(scon, [P[0], P[1], g, dinv8], [Ws[L], bs[L - 1]],
                       [Ws[L].shape[1]])[0]

    P = _sc_aggregate(g, srcp, dstp, 8)

    def sfin(p0, p1, gb, dv, bl):
        return dv[:, :1] * (p0 + p1 + gb) + bl

    out = _rows_call(sfin, [P[0], P[1], g, dinv8], [bs[11]], [8])[0]
    return out[:_N]
